# baseline (speedup 1.0000x reference)
import functools

import jax
import jax.numpy as jnp
from jax import lax
from jax.experimental import pallas as pl
from jax.experimental.pallas import tpu as pltpu
from jax.experimental.pallas import tpu_sc as plsc

CAT = 26
VOCAB = 1000
EDIM = 50
DPAD = 64
NCONT = 13
CPAD = 128
B = 4096
EPS = 1e-5

NC = 2
NS = 16
NW = NC * NS
ROWS = B * CAT
RPW = ROWS // NW
CHUNK = 128
NCHUNK = RPW // CHUNK

@functools.cache
def _make_sc_gather():
    mesh = plsc.VectorSubcoreMesh(core_axis_name="c", subcore_axis_name="s",
                                  num_cores=NC, num_subcores=NS)
    return functools.partial(
        pl.kernel,
        out_type=jax.ShapeDtypeStruct((ROWS, DPAD), jnp.bfloat16),
        mesh=mesh,
        scratch_types=[
            pltpu.VMEM((RPW,), jnp.int32),
            pltpu.VMEM((CHUNK,), jnp.int32),
            pltpu.VMEM((CHUNK, DPAD), jnp.bfloat16),
            pltpu.SemaphoreType.DMA,
        ],
        compiler_params=pltpu.CompilerParams(use_tc_tiling_on_sc=False),
    )(_sc_gather_body)


def _sc_gather_body(xcat_hbm, table_hbm, out_hbm, xcat_v, idx_v, rows_v, sem):
    wid = lax.axis_index("s") * NC + lax.axis_index("c")
    base = wid * RPW
    pltpu.sync_copy(xcat_hbm.at[pl.ds(base, RPW)], xcat_v)
    lanes = lax.iota(jnp.int32, 16)

    @pl.loop(0, NCHUNK)
    def _chunk(c):
        @pl.loop(0, CHUNK // 16)
        def _vec(j):
            p = c * CHUNK + j * 16
            xv = xcat_v[pl.ds(p, 16)]
            fld = lax.rem(p + lanes, CAT)
            idx_v[pl.ds(j * 16, 16)] = xv + VOCAB * fld

        pltpu.async_copy(table_hbm.at[idx_v], rows_v, sem).wait()
        pltpu.sync_copy(rows_v, out_hbm.at[pl.ds(base + c * CHUNK, CHUNK)])


KG = CAT * DPAD
TILE = 512


def _mlp_body(xg_ref, xc_ref, g0_ref, b0_ref,
              w1g_ref, w1c_ref, b1_ref, g1_ref, t1_ref,
              w2_ref, b2_ref, g2_ref, t2_ref,
              w3_ref, b3_ref, g3_ref, t3_ref,
              w4_ref, b4_ref, out_ref):
    cbn = 1.0 / jnp.sqrt(1.0 + EPS)
    dn = (((1,), (1,)), ((), ()))
    bf = jnp.bfloat16

    def mm(a, w_ref):
        return lax.dot_general(a.astype(bf), w_ref[:], dn,
                               preferred_element_type=jnp.float32)

    xc = xc_ref[:] * (g0_ref[:] * cbn) + b0_ref[:]
    h = mm(xg_ref[:], w1g_ref) + mm(xc, w1c_ref)
    h = jnp.maximum(h + b1_ref[:], 0.0) * (g1_ref[:] * cbn) + t1_ref[:]
    h = jnp.maximum(mm(h, w2_ref) + b2_ref[:], 0.0) * (g2_ref[:] * cbn) + t2_ref[:]
    h = jnp.maximum(mm(h, w3_ref) + b3_ref[:], 0.0) * (g3_ref[:] * cbn) + t3_ref[:]
    out_ref[:] = jnp.sum(h * w4_ref[:], axis=1, keepdims=True) + b4_ref[:]


def _full(shape):
    return pl.BlockSpec(shape, lambda i: (0, 0))


def kernel(x_cat, x_cont, emb, g0, b0, W1, bias1, g1, bt1, W2, bias2, g2, bt2,
           W3, bias3, g3, bt3, W4, bias4):
    f32 = jnp.float32
    xcat_flat = x_cat.astype(jnp.int32).reshape(ROWS)
    table = jnp.pad(emb.astype(jnp.bfloat16),
                    ((0, 0), (0, 0), (0, DPAD - EDIM))).reshape(CAT * VOCAB, DPAD)

    xg = _make_sc_gather()(xcat_flat, table).reshape(B, KG)

    xcp = jnp.pad(x_cont, ((0, 0), (0, CPAD - NCONT)))
    g0p = jnp.pad(g0, (0, CPAD - NCONT)).reshape(1, CPAD)
    b0p = jnp.pad(b0, (0, CPAD - NCONT)).reshape(1, CPAD)
    bf = jnp.bfloat16
    w1g = jnp.pad(W1[:, :CAT * EDIM].reshape(-1, CAT, EDIM),
                  ((0, 0), (0, 0), (0, DPAD - EDIM))).reshape(-1, KG).astype(bf)
    w1c = jnp.pad(W1[:, CAT * EDIM:], ((0, 0), (0, CPAD - NCONT))).astype(bf)
    H1, H2, H3 = W1.shape[0], W2.shape[0], W3.shape[0]

    row = lambda v: v.reshape(1, -1)
    out = pl.pallas_call(
        _mlp_body,
        grid=(B // TILE,),
        in_specs=[
            pl.BlockSpec((TILE, KG), lambda i: (i, 0)),
            pl.BlockSpec((TILE, CPAD), lambda i: (i, 0)),
            _full((1, CPAD)), _full((1, CPAD)),
            _full((H1, KG)), _full((H1, CPAD)),
            _full((1, H1)), _full((1, H1)), _full((1, H1)),
            _full((H2, H1)), _full((1, H2)), _full((1, H2)), _full((1, H2)),
            _full((H3, H2)), _full((1, H3)), _full((1, H3)), _full((1, H3)),
            _full((1, H3)), _full((1, 1)),
        ],
        out_specs=pl.BlockSpec((TILE, 1), lambda i: (i, 0)),
        out_shape=jax.ShapeDtypeStruct((B, 1), f32),
    )(xg, xcp, g0p, b0p,
      w1g, w1c, row(bias1), row(g1), row(bt1),
      W2.astype(bf), row(bias2), row(g2), row(bt2),
      W3.astype(bf), row(bias3), row(g3), row(bt3),
      W4, bias4.reshape(1, 1))
    return out


# speedup vs baseline: 1.2257x; 1.2257x over previous
import functools

import jax
import jax.numpy as jnp
from jax import lax
from jax.experimental import pallas as pl
from jax.experimental.pallas import tpu as pltpu
from jax.experimental.pallas import tpu_sc as plsc

CAT = 26
VOCAB = 1000
EDIM = 50
DPAD = 64
NCONT = 13
B = 4096
EPS = 1e-5

NC = 2
NS = 16
NW = NC * NS
ROWS = B * CAT
RPW = ROWS // NW
CHUNK = 128
NCHUNK = RPW // CHUNK
KG = CAT * DPAD


@functools.cache
def _make_sc_gather():
    mesh = plsc.VectorSubcoreMesh(core_axis_name="c", subcore_axis_name="s",
                                  num_cores=NC, num_subcores=NS)
    return functools.partial(
        pl.kernel,
        out_type=jax.ShapeDtypeStruct((ROWS, DPAD), jnp.float32),
        mesh=mesh,
        scratch_types=[
            pltpu.VMEM((RPW,), jnp.int32),
            pltpu.VMEM((2, CHUNK), jnp.int32),
            pltpu.VMEM((2, CHUNK, DPAD), jnp.float32),
            pltpu.SemaphoreType.DMA((2,)),
            pltpu.SemaphoreType.DMA((2,)),
        ],
        compiler_params=pltpu.CompilerParams(use_tc_tiling_on_sc=False),
    )(_sc_gather_body)


def _sc_gather_body(xcat_hbm, table_hbm, out_hbm, xcat_v, idx_v, rows_v,
                    gsem, osem):
    wid = lax.axis_index("s") * NC + lax.axis_index("c")
    base = wid * RPW
    pltpu.sync_copy(xcat_hbm.at[pl.ds(base, RPW)], xcat_v)
    lanes = lax.iota(jnp.int32, 16)

    def compute_idx(c, p):
        @pl.loop(0, CHUNK // 16)
        def _vec(j):
            pos = c * CHUNK + j * 16
            xv = xcat_v[pl.ds(pos, 16)]
            fld = lax.rem(pos + lanes, CAT)
            idx_v[p, pl.ds(j * 16, 16)] = xv + VOCAB * fld

    def gather(c, p):
        pltpu.async_copy(table_hbm.at[idx_v.at[p]], rows_v.at[p], gsem.at[p])

    def gather_wait(p):
        pltpu.make_async_copy(table_hbm.at[idx_v.at[p]], rows_v.at[p],
                              gsem.at[p]).wait()

    def out_slice(c):
        return out_hbm.at[pl.ds(base + c * CHUNK, CHUNK)]

    def copy_out(c, p):
        pltpu.async_copy(rows_v.at[p], out_slice(c), osem.at[p])

    def copy_out_wait(c, p):
        pltpu.make_async_copy(rows_v.at[p], out_slice(c), osem.at[p]).wait()

    compute_idx(0, 0)
    gather(0, 0)

    @pl.loop(0, NCHUNK)
    def _chunk(c):
        p = lax.rem(c, 2)
        q = 1 - p
        gather_wait(p)
        copy_out(c, p)

        @pl.when(c + 1 < NCHUNK)
        def _next():
            compute_idx(c + 1, q)

            @pl.when(c >= 1)
            def _free():
                copy_out_wait(c - 1, q)

            gather(c + 1, q)

    copy_out_wait(NCHUNK - 1, lax.rem(NCHUNK - 1, 2))
    copy_out_wait(NCHUNK - 2, lax.rem(NCHUNK - 2, 2))


TILE = 512


def _mlp_body(xg_ref, xc_ref, g0_ref, b0_ref,
              w1g_ref, w1c_ref, b1_ref, g1_ref, t1_ref,
              w2_ref, b2_ref, g2_ref, t2_ref,
              w3_ref, b3_ref, g3_ref, t3_ref,
              w4_ref, b4_ref, out_ref):
    cbn = 1.0 / jnp.sqrt(1.0 + EPS)
    dn = (((1,), (1,)), ((), ()))
    bf = jnp.bfloat16

    def mm(a, w):
        return lax.dot_general(a.astype(bf), w, dn,
                               preferred_element_type=jnp.float32)

    xc = xc_ref[:] * (g0_ref[:] * cbn) + b0_ref[:]
    h = mm(xg_ref[:], w1g_ref[:]) + mm(xc, w1c_ref[:])
    h = jnp.maximum(h + b1_ref[:], 0.0) * (g1_ref[:] * cbn) + t1_ref[:]
    h = jnp.maximum(mm(h, w2_ref[:]) + b2_ref[:], 0.0) * (g2_ref[:] * cbn) + t2_ref[:]
    h = jnp.maximum(mm(h, w3_ref[:]) + b3_ref[:], 0.0) * (g3_ref[:] * cbn) + t3_ref[:]
    out_ref[:] = jnp.sum(h * w4_ref[:], axis=1, keepdims=True) + b4_ref[:]


def _full(shape):
    return pl.BlockSpec(shape, lambda i: (0, 0))


def kernel(x_cat, x_cont, emb, g0, b0, W1, bias1, g1, bt1, W2, bias2, g2, bt2,
           W3, bias3, g3, bt3, W4, bias4):
    f32 = jnp.float32
    bf = jnp.bfloat16
    xcat_flat = x_cat.astype(jnp.int32).reshape(ROWS)
    table = jnp.pad(emb, ((0, 0), (0, 0), (0, DPAD - EDIM))).reshape(CAT * VOCAB, DPAD)

    xg = _make_sc_gather()(xcat_flat, table).reshape(B, KG)

    w1g = jnp.pad(W1[:, :CAT * EDIM].reshape(-1, CAT, EDIM),
                  ((0, 0), (0, 0), (0, DPAD - EDIM))).reshape(-1, KG).astype(bf)
    w1c = W1[:, CAT * EDIM:].astype(bf)
    H1, H2, H3 = W1.shape[0], W2.shape[0], W3.shape[0]

    row = lambda v: v.reshape(1, -1)
    out = pl.pallas_call(
        _mlp_body,
        grid=(B // TILE,),
        in_specs=[
            pl.BlockSpec((TILE, KG), lambda i: (i, 0)),
            pl.BlockSpec((TILE, NCONT), lambda i: (i, 0)),
            _full((1, NCONT)), _full((1, NCONT)),
            _full((H1, KG)), _full((H1, NCONT)),
            _full((1, H1)), _full((1, H1)), _full((1, H1)),
            _full((H2, H1)), _full((1, H2)), _full((1, H2)), _full((1, H2)),
            _full((H3, H2)), _full((1, H3)), _full((1, H3)), _full((1, H3)),
            _full((1, H3)), _full((1, 1)),
        ],
        out_specs=pl.BlockSpec((TILE, 1), lambda i: (i, 0)),
        out_shape=jax.ShapeDtypeStruct((B, 1), f32),
    )(xg, x_cont, row(g0), row(b0),
      w1g, w1c, row(bias1), row(g1), row(bt1),
      W2.astype(bf), row(bias2), row(g2), row(bt2),
      W3.astype(bf), row(bias3), row(g3), row(bt3),
      W4, bias4.reshape(1, 1))
    return out
